# baseline (speedup 1.0000x reference)
import functools

import jax
import jax.numpy as jnp
import numpy as np
from jax import lax
from jax.experimental import pallas as pl
from jax.experimental.pallas import tpu as pltpu
from jax.experimental.pallas import tpu_sc as plsc

BATCH = 64
SEQ = 2048
D = 128
HALF = D // 2
LANES = 16
NUM_WORKERS = 32
POS_PER_W = SEQ // NUM_WORKERS
SCALE = float(np.sqrt(float(D)))
NBUF = 4
NCHUNK = BATCH // 2


def _trig_tables():
    omega = 1.0 / (10000.0 ** (np.arange(HALF, dtype=np.float64) / HALF))
    k = np.arange(POS_PER_W, dtype=np.float64)[:, None]
    aux = np.concatenate(
        [np.cos(k * omega), np.sin(k * omega)], axis=-1)
    base = (np.arange(NUM_WORKERS, dtype=np.float64) * POS_PER_W)[:, None]
    basis = np.concatenate(
        [np.sin(base * omega), np.cos(base * omega)], axis=-1)
    return aux.astype(np.float32), basis.astype(np.float32)


_AUX, _BASIS = _trig_tables()


def _sc_kernel(x_hbm, table_hbm, aux_hbm, basis_hbm, out_hbm,
               idx_v, b0, b1, b2, b3, pe_v, aux_v, basis_v,
               g0, g1, g2, g3, s0, s1, s2, s3):
    nc = 2
    wid = lax.axis_index("s") * nc + lax.axis_index("c")
    pos_base = wid * POS_PER_W

    bufs = (b0, b1, b2, b3)
    gsems = (g0, g1, g2, g3)
    ssems = (s0, s1, s2, s3)

    def idx_copy(b):
        return pltpu.make_async_copy(
            x_hbm.at[b, pl.ds(pos_base, POS_PER_W)], idx_v.at[b], s0)

    def issue_idx(b, _):
        idx_copy(b).start()
        return 0

    def drain_idx(b, _):
        idx_copy(b).wait()
        return 0

    def gather_copies(c, p):
        buf = bufs[p]
        return (
            pltpu.make_async_copy(
                table_hbm.at[idx_v.at[2 * c]],
                buf.at[pl.ds(0, POS_PER_W)], gsems[p]),
            pltpu.make_async_copy(
                table_hbm.at[idx_v.at[2 * c + 1]],
                buf.at[pl.ds(POS_PER_W, POS_PER_W)], gsems[p]),
        )

    def scatter_copies(c, p):
        buf = bufs[p]
        return (
            pltpu.make_async_copy(
                buf.at[pl.ds(0, POS_PER_W)],
                out_hbm.at[2 * c, pl.ds(pos_base, POS_PER_W)], ssems[p]),
            pltpu.make_async_copy(
                buf.at[pl.ds(POS_PER_W, POS_PER_W)],
                out_hbm.at[2 * c + 1, pl.ds(pos_base, POS_PER_W)], ssems[p]),
        )

    def start(copies):
        for cp in copies:
            cp.start()

    def wait(copies):
        for cp in copies:
            cp.wait()

    lax.fori_loop(0, 4, issue_idx, 0)
    lax.fori_loop(0, 4, drain_idx, 0)
    start(gather_copies(0, 0))
    start(gather_copies(1, 1))

    lax.fori_loop(4, BATCH, issue_idx, 0)
    pltpu.sync_copy(aux_hbm, aux_v)
    pltpu.sync_copy(basis_hbm.at[wid], basis_v)

    def pe_row(k, _):
        for g in range(HALF // LANES):
            sl_s = pl.ds(g * LANES, LANES)
            sl_c = pl.ds(HALF + g * LANES, LANES)
            kc = aux_v[k, sl_s]
            ks = aux_v[k, sl_c]
            sb = basis_v[sl_s]
            cb = basis_v[sl_c]
            pe_v[k, sl_s] = sb * kc + cb * ks
            pe_v[k, sl_c] = cb * kc - sb * ks
        return 0
    lax.fori_loop(0, POS_PER_W, pe_row, 0)
    lax.fori_loop(4, BATCH, drain_idx, 0)

    def compute(buf):
        def row_body(r, _):
            for j in range(D // LANES):
                sl = pl.ds(j * LANES, LANES)
                pv = pe_v[r, sl]
                buf[r, sl] = buf[r, sl] * SCALE + pv
                r2 = r + POS_PER_W
                buf[r2, sl] = buf[r2, sl] * SCALE + pv
            return 0
        lax.fori_loop(0, POS_PER_W, row_body, 0)

    def step(c0, _):
        for p in range(NBUF):
            c = c0 + p
            wait(gather_copies(c, p))
            compute(bufs[p])
            start(scatter_copies(c, p))

            @pl.when(c + 2 < NCHUNK)
            def _():
                pn = (p + 2) % NBUF

                @pl.when(c - 2 >= 0)
                def _():
                    wait(scatter_copies(c - 2, pn))
                start(gather_copies(c + 2, pn))
        return 0

    lax.fori_loop(0, NCHUNK // NBUF, lambda i, cr: step(i * NBUF, cr), 0)

    for c in range(NCHUNK - 4, NCHUNK):
        wait(scatter_copies(c, c % NBUF))


def kernel(x, table):
    x = x.astype(jnp.int32)
    aux = jnp.asarray(_AUX)
    basis = jnp.asarray(_BASIS)
    mesh = plsc.VectorSubcoreMesh(core_axis_name="c", subcore_axis_name="s")
    k = functools.partial(
        pl.kernel,
        mesh=mesh,
        out_type=jax.ShapeDtypeStruct((BATCH, SEQ, D), jnp.float32),
        scratch_types=[
            pltpu.VMEM((BATCH, POS_PER_W), jnp.int32),
            pltpu.VMEM((2 * POS_PER_W, D), jnp.float32),
            pltpu.VMEM((2 * POS_PER_W, D), jnp.float32),
            pltpu.VMEM((2 * POS_PER_W, D), jnp.float32),
            pltpu.VMEM((2 * POS_PER_W, D), jnp.float32),
            pltpu.VMEM((POS_PER_W, D), jnp.float32),
            pltpu.VMEM((POS_PER_W, D), jnp.float32),
            pltpu.VMEM((D,), jnp.float32),
            pltpu.SemaphoreType.DMA,
            pltpu.SemaphoreType.DMA,
            pltpu.SemaphoreType.DMA,
            pltpu.SemaphoreType.DMA,
            pltpu.SemaphoreType.DMA,
            pltpu.SemaphoreType.DMA,
            pltpu.SemaphoreType.DMA,
            pltpu.SemaphoreType.DMA,
        ],
    )(_sc_kernel)
    return k(x, table, aux, basis)


# speedup vs baseline: 1.0495x; 1.0495x over previous
import functools

import jax
import jax.numpy as jnp
import numpy as np
from jax import lax
from jax.experimental import pallas as pl
from jax.experimental.pallas import tpu as pltpu
from jax.experimental.pallas import tpu_sc as plsc

BATCH = 64
SEQ = 2048
D = 128
LANES = 16
NUM_WORKERS = 32
POS_PER_W = SEQ // NUM_WORKERS
SCALE = float(np.sqrt(float(D)))
NBUF = 4
NCHUNK = BATCH // 2


def _positional_encoding(length, depth):
    half = depth // 2
    positions = np.arange(length)[:, None].astype(np.float32)
    depths = (np.arange(half)[None, :] / float(half)).astype(np.float32)
    angle_rates = 1.0 / (10000.0 ** depths)
    angle_rads = positions * angle_rates
    return np.concatenate([np.sin(angle_rads), np.cos(angle_rads)], axis=-1)


_PE = _positional_encoding(SEQ, D)


def _sc_kernel(x_hbm, table_hbm, pe_hbm, out_hbm,
               idx_v, b0, b1, b2, b3, pe_v,
               g0, g1, g2, g3, s0, s1, s2, s3):
    nc = 2
    wid = lax.axis_index("s") * nc + lax.axis_index("c")
    pos_base = wid * POS_PER_W

    bufs = (b0, b1, b2, b3)
    gsems = (g0, g1, g2, g3)
    ssems = (s0, s1, s2, s3)

    def idx_copy(b):
        off = pl.multiple_of(b * POS_PER_W, POS_PER_W)
        return pltpu.make_async_copy(
            x_hbm.at[b, pl.ds(pos_base, POS_PER_W)],
            idx_v.at[pl.ds(off, POS_PER_W)], s0)

    def issue_idx(b, _):
        idx_copy(b).start()
        return 0

    def drain_idx(b, _):
        idx_copy(b).wait()
        return 0

    lax.fori_loop(0, 4, issue_idx, 0)
    lax.fori_loop(0, 4, drain_idx, 0)

    def gather_copies(c, p):
        off = pl.multiple_of(c * 2 * POS_PER_W, 2 * POS_PER_W)
        return (
            pltpu.make_async_copy(
                table_hbm.at[idx_v.at[pl.ds(off, 2 * POS_PER_W)]],
                bufs[p], gsems[p]),
        )

    def scatter_copies(c, p):
        buf = bufs[p]
        return (
            pltpu.make_async_copy(
                buf.at[pl.ds(0, POS_PER_W)],
                out_hbm.at[2 * c, pl.ds(pos_base, POS_PER_W)], ssems[p]),
            pltpu.make_async_copy(
                buf.at[pl.ds(POS_PER_W, POS_PER_W)],
                out_hbm.at[2 * c + 1, pl.ds(pos_base, POS_PER_W)], ssems[p]),
        )

    def start(copies):
        for cp in copies:
            cp.start()

    def wait(copies):
        for cp in copies:
            cp.wait()

    def compute(buf):
        def row_body(r, _):
            for j in range(D // LANES):
                sl = pl.ds(j * LANES, LANES)
                pv = pe_v[r, sl]
                buf[r, sl] = buf[r, sl] * SCALE + pv
                r2 = r + POS_PER_W
                buf[r2, sl] = buf[r2, sl] * SCALE + pv
            return 0
        lax.fori_loop(0, POS_PER_W, row_body, 0)

    start(gather_copies(0, 0))
    start(gather_copies(1, 1))
    lax.fori_loop(4, BATCH, issue_idx, 0)
    pltpu.sync_copy(pe_hbm.at[pl.ds(pos_base, POS_PER_W)], pe_v)
    lax.fori_loop(4, BATCH, drain_idx, 0)

    def step(c0, _):
        for p in range(NBUF):
            c = c0 + p
            wait(gather_copies(c, p))
            compute(bufs[p])
            start(scatter_copies(c, p))

            @pl.when(c + 2 < NCHUNK)
            def _():
                pn = (p + 2) % NBUF

                @pl.when(c - 2 >= 0)
                def _():
                    wait(scatter_copies(c - 2, pn))
                start(gather_copies(c + 2, pn))
        return 0

    lax.fori_loop(0, NCHUNK // NBUF, lambda i, cr: step(i * NBUF, cr), 0)

    for c in range(NCHUNK - 4, NCHUNK):
        wait(scatter_copies(c, c % NBUF))


def kernel(x, table):
    x = x.astype(jnp.int32)
    pe = jnp.asarray(_PE, dtype=jnp.float32)
    mesh = plsc.VectorSubcoreMesh(core_axis_name="c", subcore_axis_name="s")
    k = functools.partial(
        pl.kernel,
        mesh=mesh,
        out_type=jax.ShapeDtypeStruct((BATCH, SEQ, D), jnp.float32),
        scratch_types=[
            pltpu.VMEM((BATCH * POS_PER_W,), jnp.int32),
            pltpu.VMEM((2 * POS_PER_W, D), jnp.float32),
            pltpu.VMEM((2 * POS_PER_W, D), jnp.float32),
            pltpu.VMEM((2 * POS_PER_W, D), jnp.float32),
            pltpu.VMEM((2 * POS_PER_W, D), jnp.float32),
            pltpu.VMEM((POS_PER_W, D), jnp.float32),
            pltpu.SemaphoreType.DMA,
            pltpu.SemaphoreType.DMA,
            pltpu.SemaphoreType.DMA,
            pltpu.SemaphoreType.DMA,
            pltpu.SemaphoreType.DMA,
            pltpu.SemaphoreType.DMA,
            pltpu.SemaphoreType.DMA,
            pltpu.SemaphoreType.DMA,
        ],
    )(_sc_kernel)
    return k(x, table, pe)


# speedup vs baseline: 1.0564x; 1.0066x over previous
import functools

import jax
import jax.numpy as jnp
import numpy as np
from jax import lax
from jax.experimental import pallas as pl
from jax.experimental.pallas import tpu as pltpu
from jax.experimental.pallas import tpu_sc as plsc

BATCH = 64
SEQ = 2048
D = 128
LANES = 16
NUM_WORKERS = 32
POS_PER_W = SEQ // NUM_WORKERS
SCALE = float(np.sqrt(float(D)))
NBUF = 3
CHUNK_B = 4
NCHUNK = BATCH // CHUNK_B
ROWS = CHUNK_B * POS_PER_W


def _positional_encoding(length, depth):
    half = depth // 2
    positions = np.arange(length)[:, None].astype(np.float32)
    depths = (np.arange(half)[None, :] / float(half)).astype(np.float32)
    angle_rates = 1.0 / (10000.0 ** depths)
    angle_rads = positions * angle_rates
    return np.concatenate([np.sin(angle_rads), np.cos(angle_rads)], axis=-1)


_PE = _positional_encoding(SEQ, D)


def _sc_kernel(x_hbm, table_hbm, pe_hbm, out_hbm,
               idx_v, b0, b1, b2, pe_v,
               g0, g1, g2, s0, s1, s2):
    nc = 2
    wid = lax.axis_index("s") * nc + lax.axis_index("c")
    pos_base = wid * POS_PER_W

    bufs = (b0, b1, b2)
    gsems = (g0, g1, g2)
    ssems = (s0, s1, s2)

    def idx_copy(b):
        off = pl.multiple_of(b * POS_PER_W, POS_PER_W)
        return pltpu.make_async_copy(
            x_hbm.at[b, pl.ds(pos_base, POS_PER_W)],
            idx_v.at[pl.ds(off, POS_PER_W)], s0)

    def issue_idx(b, _):
        idx_copy(b).start()
        return 0

    def drain_idx(b, _):
        idx_copy(b).wait()
        return 0

    def gather_copies(c, p):
        buf = bufs[p]
        off = pl.multiple_of(c * ROWS, ROWS)
        off2 = pl.multiple_of(c * ROWS + ROWS // 2, ROWS // 2)
        return (
            pltpu.make_async_copy(
                table_hbm.at[idx_v.at[pl.ds(off, ROWS // 2)]],
                buf.at[pl.ds(0, ROWS // 2)], gsems[p]),
            pltpu.make_async_copy(
                table_hbm.at[idx_v.at[pl.ds(off2, ROWS // 2)]],
                buf.at[pl.ds(ROWS // 2, ROWS // 2)], gsems[p]),
        )

    def scatter_copies(c, p):
        buf = bufs[p]
        return tuple(
            pltpu.make_async_copy(
                buf.at[pl.ds(q * POS_PER_W, POS_PER_W)],
                out_hbm.at[CHUNK_B * c + q, pl.ds(pos_base, POS_PER_W)],
                ssems[p])
            for q in range(CHUNK_B)
        )

    def start(copies):
        for cp in copies:
            cp.start()

    def wait(copies):
        for cp in copies:
            cp.wait()

    lax.fori_loop(0, 2 * CHUNK_B, issue_idx, 0)
    lax.fori_loop(0, 2 * CHUNK_B, drain_idx, 0)
    start(gather_copies(0, 0))
    start(gather_copies(1, 1))
    lax.fori_loop(2 * CHUNK_B, BATCH, issue_idx, 0)
    pltpu.sync_copy(pe_hbm.at[pl.ds(pos_base, POS_PER_W)], pe_v)
    lax.fori_loop(2 * CHUNK_B, BATCH, drain_idx, 0)

    def compute(buf):
        def row_body(r, _):
            for j in range(D // LANES):
                sl = pl.ds(j * LANES, LANES)
                pv = pe_v[r, sl]
                for q in range(CHUNK_B):
                    rq = r + q * POS_PER_W
                    buf[rq, sl] = buf[rq, sl] * SCALE + pv
            return 0
        lax.fori_loop(0, POS_PER_W, row_body, 0)

    def substep(c, p):
        wait(gather_copies(c, p))
        compute(bufs[p])
        start(scatter_copies(c, p))

        @pl.when(c + 2 < NCHUNK)
        def _():
            pn = (p + 2) % NBUF

            @pl.when(c - 1 >= 0)
            def _():
                wait(scatter_copies(c - 1, pn))
            start(gather_copies(c + 2, pn))

    def step(c0, _):
        for p in range(NBUF):
            substep(c0 + p, p)
        return 0

    lax.fori_loop(0, (NCHUNK - 1) // NBUF, lambda i, cr: step(i * NBUF, cr), 0)
    substep(NCHUNK - 1, (NCHUNK - 1) % NBUF)

    for c in range(NCHUNK - 3, NCHUNK):
        wait(scatter_copies(c, c % NBUF))


def kernel(x, table):
    x = x.astype(jnp.int32)
    pe = jnp.asarray(_PE, dtype=jnp.float32)
    mesh = plsc.VectorSubcoreMesh(core_axis_name="c", subcore_axis_name="s")
    k = functools.partial(
        pl.kernel,
        mesh=mesh,
        out_type=jax.ShapeDtypeStruct((BATCH, SEQ, D), jnp.float32),
        scratch_types=[
            pltpu.VMEM((BATCH * POS_PER_W,), jnp.int32),
            pltpu.VMEM((ROWS, D), jnp.float32),
            pltpu.VMEM((ROWS, D), jnp.float32),
            pltpu.VMEM((ROWS, D), jnp.float32),
            pltpu.VMEM((POS_PER_W, D), jnp.float32),
            pltpu.SemaphoreType.DMA,
            pltpu.SemaphoreType.DMA,
            pltpu.SemaphoreType.DMA,
            pltpu.SemaphoreType.DMA,
            pltpu.SemaphoreType.DMA,
            pltpu.SemaphoreType.DMA,
        ],
    )(_sc_kernel)
    return k(x, table, pe)


# speedup vs baseline: 1.0667x; 1.0098x over previous
import functools

import jax
import jax.numpy as jnp
import numpy as np
from jax import lax
from jax.experimental import pallas as pl
from jax.experimental.pallas import tpu as pltpu
from jax.experimental.pallas import tpu_sc as plsc

BATCH = 64
SEQ = 2048
D = 128
LANES = 16
NUM_WORKERS = 32
POS_PER_W = SEQ // NUM_WORKERS
SCALE = float(np.sqrt(float(D)))
NBUF = 3
CHUNK_B = 4
NCHUNK = BATCH // CHUNK_B
ROWS = CHUNK_B * POS_PER_W


def _positional_encoding(length, depth):
    half = depth // 2
    positions = np.arange(length)[:, None].astype(np.float32)
    depths = (np.arange(half)[None, :] / float(half)).astype(np.float32)
    angle_rates = 1.0 / (10000.0 ** depths)
    angle_rads = positions * angle_rates
    return np.concatenate([np.sin(angle_rads), np.cos(angle_rads)], axis=-1)


_PE = _positional_encoding(SEQ, D)


def _sc_kernel(x_hbm, table_hbm, pe_hbm, out_hbm,
               idx_v, b0, b1, b2, pe_v,
               g0, g1, g2, s0, s1, s2):
    nc = 2
    wid = lax.axis_index("s") * nc + lax.axis_index("c")
    pos_base = wid * POS_PER_W

    bufs = (b0, b1, b2)
    gsems = (g0, g1, g2)
    ssems = (s0, s1, s2)

    def idx_copy(b):
        off = pl.multiple_of(b * POS_PER_W, POS_PER_W)
        return pltpu.make_async_copy(
            x_hbm.at[b, pl.ds(pos_base, POS_PER_W)],
            idx_v.at[pl.ds(off, POS_PER_W)], s0)

    def issue_idx(b, _):
        idx_copy(b).start()
        return 0

    def drain_idx(b, _):
        idx_copy(b).wait()
        return 0

    def gather_copies(c, p):
        buf = bufs[p]
        off = pl.multiple_of(c * ROWS, ROWS)
        off2 = pl.multiple_of(c * ROWS + ROWS // 2, ROWS // 2)
        return (
            pltpu.make_async_copy(
                table_hbm.at[idx_v.at[pl.ds(off, ROWS // 2)]],
                buf.at[pl.ds(0, ROWS // 2)], gsems[p]),
            pltpu.make_async_copy(
                table_hbm.at[idx_v.at[pl.ds(off2, ROWS // 2)]],
                buf.at[pl.ds(ROWS // 2, ROWS // 2)], gsems[p]),
        )

    HALF_P = POS_PER_W // 2

    def scatter_half(c, p, h):
        buf = bufs[p]
        return tuple(
            pltpu.make_async_copy(
                buf.at[pl.ds(q * POS_PER_W + h * HALF_P, HALF_P)],
                out_hbm.at[CHUNK_B * c + q,
                           pl.ds(pos_base + h * HALF_P, HALF_P)],
                ssems[p])
            for q in range(CHUNK_B)
        )

    def scatter_copies(c, p):
        return scatter_half(c, p, 0) + scatter_half(c, p, 1)

    def start(copies):
        for cp in copies:
            cp.start()

    def wait(copies):
        for cp in copies:
            cp.wait()

    lax.fori_loop(0, 2 * CHUNK_B, issue_idx, 0)
    lax.fori_loop(0, 2 * CHUNK_B, drain_idx, 0)
    start(gather_copies(0, 0))
    start(gather_copies(1, 1))
    lax.fori_loop(2 * CHUNK_B, BATCH, issue_idx, 0)
    pltpu.sync_copy(pe_hbm.at[pl.ds(pos_base, POS_PER_W)], pe_v)
    lax.fori_loop(2 * CHUNK_B, BATCH, drain_idx, 0)

    def compute_half(buf, h):
        def row_body(r, _):
            for j in range(D // LANES):
                sl = pl.ds(j * LANES, LANES)
                pv = pe_v[r, sl]
                for q in range(CHUNK_B):
                    rq = r + q * POS_PER_W
                    buf[rq, sl] = buf[rq, sl] * SCALE + pv
            return 0
        lax.fori_loop(h * HALF_P, (h + 1) * HALF_P, row_body, 0)

    def substep(c, p):
        wait(gather_copies(c, p))
        compute_half(bufs[p], 0)
        start(scatter_half(c, p, 0))
        compute_half(bufs[p], 1)
        start(scatter_half(c, p, 1))

        @pl.when(c + 2 < NCHUNK)
        def _():
            pn = (p + 2) % NBUF

            @pl.when(c - 1 >= 0)
            def _():
                wait(scatter_copies(c - 1, pn))
            start(gather_copies(c + 2, pn))

    def step(c0, _):
        for p in range(NBUF):
            substep(c0 + p, p)
        return 0

    lax.fori_loop(0, (NCHUNK - 1) // NBUF, lambda i, cr: step(i * NBUF, cr), 0)
    substep(NCHUNK - 1, (NCHUNK - 1) % NBUF)

    for c in range(NCHUNK - 3, NCHUNK):
        wait(scatter_copies(c, c % NBUF))


def kernel(x, table):
    x = x.astype(jnp.int32)
    pe = jnp.asarray(_PE, dtype=jnp.float32)
    mesh = plsc.VectorSubcoreMesh(core_axis_name="c", subcore_axis_name="s")
    k = functools.partial(
        pl.kernel,
        mesh=mesh,
        out_type=jax.ShapeDtypeStruct((BATCH, SEQ, D), jnp.float32),
        scratch_types=[
            pltpu.VMEM((BATCH * POS_PER_W,), jnp.int32),
            pltpu.VMEM((ROWS, D), jnp.float32),
            pltpu.VMEM((ROWS, D), jnp.float32),
            pltpu.VMEM((ROWS, D), jnp.float32),
            pltpu.VMEM((POS_PER_W, D), jnp.float32),
            pltpu.SemaphoreType.DMA,
            pltpu.SemaphoreType.DMA,
            pltpu.SemaphoreType.DMA,
            pltpu.SemaphoreType.DMA,
            pltpu.SemaphoreType.DMA,
            pltpu.SemaphoreType.DMA,
        ],
    )(_sc_kernel)
    return k(x, table, pe)
